# baseline (speedup 1.0000x reference)
import functools

import jax
import jax.numpy as jnp
from jax import lax
from jax.experimental import pallas as pl
from jax.experimental.pallas import tpu as pltpu
from jax.experimental.pallas import tpu_sc as plsc

_NBUF = 8


def _gather_kernel(NB, S, D, BPC):
    SP = (S + 7) // 8 * 8
    DP = (D + 127) // 128 * 128
    CH = BPC * S
    info = plsc.get_sparse_core_info()
    NC, NS = info.num_cores, info.num_subcores
    NW = NC * NS
    nb_per_w = NB // NW
    n_ch = nb_per_w // BPC
    n_outer = n_ch // _NBUF
    assert nb_per_w % BPC == 0 and n_ch % _NBUF == 0 and n_outer >= 3
    mesh = plsc.VectorSubcoreMesh(core_axis_name="c", subcore_axis_name="s")

    @functools.partial(
        pl.kernel,
        mesh=mesh,
        out_type=jax.ShapeDtypeStruct((NB, SP, DP), jnp.float32),
        scratch_types=[
            pltpu.VMEM((n_ch, CH), jnp.int32),
            pltpu.VMEM((_NBUF, CH, D), jnp.float32),
            pltpu.SemaphoreType.DMA((_NBUF,)),
            pltpu.SemaphoreType.DMA((_NBUF,)),
        ],
        compiler_params=pltpu.CompilerParams(use_tc_tiling_on_sc=False),
    )
    def k(idx_hbm, table_hbm, out_hbm, idx_v, rows_v, sem_g, sem_o):
        wid = lax.axis_index("s") * NC + lax.axis_index("c")
        c0 = wid * n_ch

        pltpu.sync_copy(idx_hbm.at[pl.ds(c0, n_ch)], idx_v)

        def gat(i, b):
            return pltpu.make_async_copy(
                table_hbm.at[idx_v.at[i]], rows_v.at[b], sem_g.at[b])

        def _sto(i, b, kq):
            return pltpu.make_async_copy(
                rows_v.at[b, pl.ds(kq * S, S)],
                out_hbm.at[(c0 + i) * BPC + kq, pl.ds(0, S), pl.ds(0, D)],
                sem_o.at[b])

        def sto_start(i, b):
            for kq in range(BPC):
                _sto(i, b, kq).start()

        def sto_wait(i, b):
            for kq in range(BPC):
                _sto(i, b, kq).wait()

        for b in range(_NBUF - 1):
            gat(b, b).start()

        for b in range(_NBUF):
            gat(b, b).wait()
            sto_start(b, b)
            if b > 0:
                sto_wait(b - 1, b - 1)
            gat(b + _NBUF - 1, (b + _NBUF - 1) % _NBUF).start()

        def outer(g, carry):
            i0 = g * _NBUF
            for b in range(_NBUF):
                i = i0 + b
                gat(i, b).wait()
                sto_start(i, b)
                bp = (b - 1) % _NBUF
                sto_wait(i - 1, bp)
                gat(i + _NBUF - 1, bp).start()
            return carry

        lax.fori_loop(1, n_outer - 1, outer, 0)

        i0 = (n_outer - 1) * _NBUF
        for b in range(_NBUF):
            i = i0 + b
            gat(i, b).wait()
            sto_start(i, b)
            bp = (b - 1) % _NBUF
            sto_wait(i - 1, bp)
            if b == 0:
                gat(i + _NBUF - 1, bp).start()
        sto_wait(i0 + _NBUF - 1, _NBUF - 1)

    return k


def kernel(element, table):
    NB, S = element.shape
    V, D = table.shape
    BPC = 2
    idx = element.reshape(NB // BPC, BPC * S)
    padded = _gather_kernel(NB, S, D, BPC)(idx, table)
    return padded[:, :S, :D]


# speedup vs baseline: 1.0032x; 1.0032x over previous
import functools

import jax
import jax.numpy as jnp
from jax import lax
from jax.experimental import pallas as pl
from jax.experimental.pallas import tpu as pltpu
from jax.experimental.pallas import tpu_sc as plsc

_NBUF = 4


def _gather_kernel(NB, S, D, BPC):
    SP = (S + 7) // 8 * 8
    DP = (D + 127) // 128 * 128
    CH = BPC * S
    info = plsc.get_sparse_core_info()
    NC, NS = info.num_cores, info.num_subcores
    NW = NC * NS
    nb_per_w = NB // NW
    n_ch = nb_per_w // BPC
    n_outer = n_ch // _NBUF
    assert nb_per_w % BPC == 0 and n_ch % _NBUF == 0 and n_outer >= 3
    mesh = plsc.VectorSubcoreMesh(core_axis_name="c", subcore_axis_name="s")

    @functools.partial(
        pl.kernel,
        mesh=mesh,
        out_type=jax.ShapeDtypeStruct((NB, SP, DP), jnp.float32),
        scratch_types=[
            pltpu.VMEM((n_ch, CH), jnp.int32),
            pltpu.VMEM((_NBUF, CH, D), jnp.float32),
            pltpu.SemaphoreType.DMA((_NBUF,)),
            pltpu.SemaphoreType.DMA((_NBUF,)),
        ],
        compiler_params=pltpu.CompilerParams(use_tc_tiling_on_sc=False),
    )
    def k(idx_hbm, table_hbm, out_hbm, idx_v, rows_v, sem_g, sem_o):
        wid = lax.axis_index("s") * NC + lax.axis_index("c")
        c0 = wid * n_ch

        pltpu.sync_copy(idx_hbm.at[pl.ds(c0, n_ch)], idx_v)

        def gat(i, b):
            return pltpu.make_async_copy(
                table_hbm.at[idx_v.at[i]], rows_v.at[b], sem_g.at[b])

        def _sto(i, b, kq):
            return pltpu.make_async_copy(
                rows_v.at[b, pl.ds(kq * S, S)],
                out_hbm.at[(c0 + i) * BPC + kq, pl.ds(0, S), pl.ds(0, D)],
                sem_o.at[b])

        def sto_start(i, b):
            for kq in range(BPC):
                _sto(i, b, kq).start()

        def sto_wait(i, b):
            for kq in range(BPC):
                _sto(i, b, kq).wait()

        for b in range(_NBUF - 1):
            gat(b, b).start()

        for b in range(_NBUF):
            gat(b, b).wait()
            sto_start(b, b)
            if b > 0:
                sto_wait(b - 1, b - 1)
            gat(b + _NBUF - 1, (b + _NBUF - 1) % _NBUF).start()

        def outer(g, carry):
            i0 = g * _NBUF
            for b in range(_NBUF):
                i = i0 + b
                gat(i, b).wait()
                sto_start(i, b)
                bp = (b - 1) % _NBUF
                sto_wait(i - 1, bp)
                gat(i + _NBUF - 1, bp).start()
            return carry

        lax.fori_loop(1, n_outer - 1, outer, 0)

        i0 = (n_outer - 1) * _NBUF
        for b in range(_NBUF):
            i = i0 + b
            gat(i, b).wait()
            sto_start(i, b)
            bp = (b - 1) % _NBUF
            sto_wait(i - 1, bp)
            if b == 0:
                gat(i + _NBUF - 1, bp).start()
        sto_wait(i0 + _NBUF - 1, _NBUF - 1)

    return k


def kernel(element, table):
    NB, S = element.shape
    V, D = table.shape
    BPC = 4
    idx = element.reshape(NB // BPC, BPC * S)
    padded = _gather_kernel(NB, S, D, BPC)(idx, table)
    return padded[:, :S, :D]
